# baseline (speedup 1.0000x reference)
import functools

import jax
import jax.numpy as jnp
from jax import lax
from jax.experimental import pallas as pl
from jax.experimental.pallas import tpu as pltpu
from jax.experimental.pallas import tpu_sc as plsc

B_, N_, F_ = 4, 1024, 128
H_, C_ = 4, 128
HC_ = H_ * C_
K_ = 512


def _stj(astack, xp):
    return jax.lax.dot_general(astack, xp, (((0,), (1,)), ((), ())),
                               preferred_element_type=jnp.float32)


def _leaky(x):
    return jnp.maximum(x, 0.2 * x)


def _heads(maskf, st, xp, n):
    outs = []
    for h in range(H_):
        s = st[h]
        t = st[H_ + h]
        mhat = _leaky(s + jnp.max(t))
        a = (s - mhat)[:, None] + t[None, :]
        b2 = (0.2 * s - mhat)[:, None] + (0.2 * t)[None, :]
        p = jnp.exp(jnp.maximum(a, b2)) * maskf
        v = jnp.concatenate(
            [xp[:, h * C_:(h + 1) * C_], jnp.ones((n, 128), jnp.float32)],
            axis=1)
        oh = jnp.dot(p, v, preferred_element_type=jnp.float32)
        outs.append(oh[:, :C_] * (1.0 / oh[:, C_:C_ + 1]))
    return outs


def _gat_cat(maskf, st, xp, n):
    return jnp.concatenate(_heads(maskf, st, xp, n), axis=1)


def _gat_mean(maskf, st, xp, n):
    o = _heads(maskf, st, xp, n)
    return jnp.maximum((o[0] + o[1] + o[2] + o[3]) * (1.0 / H_), 0.0)


def _diag_mask(pos, n):
    r = jax.lax.broadcasted_iota(jnp.int32, (n, n), 0)
    c = jax.lax.broadcasted_iota(jnp.int32, (n, n), 1)
    return pos | (r == c)


def _enc_body(x_ref, adj_ref, w0a_ref, a0a_ref, w0b_ref, a0b_ref, pk_ref,
              m_ref, d_ref, g_ref, y_ref):
    m = _diag_mask(adj_ref[0] > 0, N_)
    m_ref[0] = m.astype(jnp.int8)
    maskf = m.astype(jnp.float32)
    xp = jnp.dot(x_ref[0], w0a_ref[...], preferred_element_type=jnp.float32)
    h0a = _gat_cat(maskf, _stj(a0a_ref[...], xp), xp, N_)
    xp = jnp.dot(h0a, w0b_ref[...], preferred_element_type=jnp.float32)
    h = _gat_mean(maskf, _stj(a0b_ref[...], xp), xp, N_)
    d_ref[0] = h
    pk = pk_ref[0]
    kn = pk / (jnp.sqrt(jnp.sum(pk * pk)) + 1e-12)
    y = jnp.sum(h * kn[None, :], axis=1)
    y_ref[0, 0, :] = y
    g_ref[0] = h * jnp.tanh(y)[:, None]


def _enc_call(X, A, w0a, a0a, w0b, a0b, pk):
    return pl.pallas_call(
        _enc_body, grid=(B_,),
        in_specs=[
            pl.BlockSpec((1, N_, F_), lambda i: (i, 0, 0)),
            pl.BlockSpec((1, N_, N_), lambda i: (i, 0, 0)),
            pl.BlockSpec((F_, HC_), lambda i: (0, 0)),
            pl.BlockSpec((HC_, 8), lambda i: (0, 0)),
            pl.BlockSpec((HC_, HC_), lambda i: (0, 0)),
            pl.BlockSpec((HC_, 8), lambda i: (0, 0)),
            pl.BlockSpec((1, C_), lambda i: (0, 0)),
        ],
        out_specs=[
            pl.BlockSpec((1, N_, N_), lambda i: (i, 0, 0)),
            pl.BlockSpec((1, N_, C_), lambda i: (i, 0, 0)),
            pl.BlockSpec((1, N_, C_), lambda i: (i, 0, 0)),
            pl.BlockSpec((1, 1, N_), lambda i: (i, 0, 0)),
        ],
        out_shape=[
            jax.ShapeDtypeStruct((B_, N_, N_), jnp.int8),
            jax.ShapeDtypeStruct((B_, N_, C_), jnp.float32),
            jax.ShapeDtypeStruct((B_, N_, C_), jnp.float32),
            jax.ShapeDtypeStruct((B_, 1, N_), jnp.float32),
        ],
    )(X, A, w0a, a0a, w0b, a0b, pk)


def _mid_body(hp_ref, ar_ref, oh_ref, w1a_ref, a1a_ref, w1b_ref, a1b_ref,
              o_ref):
    ap = jnp.dot(ar_ref[0], oh_ref[0], preferred_element_type=jnp.float32)
    maskf = _diag_mask(ap > 0, K_).astype(jnp.float32)
    xp = jnp.dot(hp_ref[0], w1a_ref[...], preferred_element_type=jnp.float32)
    h1a = _gat_cat(maskf, _stj(a1a_ref[...], xp), xp, K_)
    xp = jnp.dot(h1a, w1b_ref[...], preferred_element_type=jnp.float32)
    o_ref[0] = _gat_mean(maskf, _stj(a1b_ref[...], xp), xp, K_)


def _mid_call(hp, Ar, oh, w1a, a1a, w1b, a1b):
    return pl.pallas_call(
        _mid_body, grid=(B_,),
        in_specs=[
            pl.BlockSpec((1, K_, C_), lambda i: (i, 0, 0)),
            pl.BlockSpec((1, K_, N_), lambda i: (i, 0, 0)),
            pl.BlockSpec((1, N_, K_), lambda i: (i, 0, 0)),
            pl.BlockSpec((C_, HC_), lambda i: (0, 0)),
            pl.BlockSpec((HC_, 8), lambda i: (0, 0)),
            pl.BlockSpec((HC_, HC_), lambda i: (0, 0)),
            pl.BlockSpec((HC_, 8), lambda i: (0, 0)),
        ],
        out_specs=pl.BlockSpec((1, K_, C_), lambda i: (i, 0, 0)),
        out_shape=jax.ShapeDtypeStruct((B_, K_, C_), jnp.float32),
    )(hp, Ar, oh, w1a, a1a, w1b, a1b)


def _dec_body(h1b_ref, oh_ref, x_ref, d_ref, m_ref, wua_ref, aua_ref, wub_ref,
              aub_ref, weah_ref, weal_ref, aea_ref, web_ref, aeb_ref, o_ref):
    maskf = m_ref[0].astype(jnp.float32)
    xp = jnp.dot(oh_ref[0],
                 jnp.dot(h1b_ref[0], wua_ref[...],
                         preferred_element_type=jnp.float32),
                 preferred_element_type=jnp.float32)
    hua = _gat_cat(maskf, _stj(aua_ref[...], xp), xp, N_)
    xp = jnp.dot(hua, wub_ref[...], preferred_element_type=jnp.float32)
    hu = _gat_mean(maskf, _stj(aub_ref[...], xp), xp, N_) + d_ref[0]
    xp = jnp.dot(hu, weah_ref[...], preferred_element_type=jnp.float32)
    xp = xp + jnp.dot(x_ref[0], weal_ref[...],
                      preferred_element_type=jnp.float32)
    hea = _gat_cat(maskf, _stj(aea_ref[...], xp), xp, N_)
    xp = jnp.dot(hea, web_ref[...], preferred_element_type=jnp.float32)
    o_ref[0] = _gat_mean(maskf, _stj(aeb_ref[...], xp), xp, N_)


def _dec_call(h1b, oh, X, down, mask8, wua, aua, wub, aub, weah, weal, aea,
              web, aeb):
    return pl.pallas_call(
        _dec_body, grid=(B_,),
        in_specs=[
            pl.BlockSpec((1, K_, C_), lambda i: (i, 0, 0)),
            pl.BlockSpec((1, N_, K_), lambda i: (i, 0, 0)),
            pl.BlockSpec((1, N_, F_), lambda i: (i, 0, 0)),
            pl.BlockSpec((1, N_, C_), lambda i: (i, 0, 0)),
            pl.BlockSpec((1, N_, N_), lambda i: (i, 0, 0)),
            pl.BlockSpec((C_, HC_), lambda i: (0, 0)),
            pl.BlockSpec((HC_, 8), lambda i: (0, 0)),
            pl.BlockSpec((HC_, HC_), lambda i: (0, 0)),
            pl.BlockSpec((HC_, 8), lambda i: (0, 0)),
            pl.BlockSpec((C_, HC_), lambda i: (0, 0)),
            pl.BlockSpec((F_, HC_), lambda i: (0, 0)),
            pl.BlockSpec((HC_, 8), lambda i: (0, 0)),
            pl.BlockSpec((HC_, HC_), lambda i: (0, 0)),
            pl.BlockSpec((HC_, 8), lambda i: (0, 0)),
        ],
        out_specs=pl.BlockSpec((1, N_, C_), lambda i: (i, 0, 0)),
        out_shape=jax.ShapeDtypeStruct((B_, N_, C_), jnp.float32),
    )(h1b, oh, X, down, mask8, wua, aua, wub, aub, weah, weal, aea, web, aeb)


_NC, _NS = 2, 16


def _sc_pool(A2, hg2, idx2, eye, zrows):
    mesh = plsc.VectorSubcoreMesh(core_axis_name="c", subcore_axis_name="s")

    @functools.partial(
        pl.kernel, mesh=mesh,
        out_type=[jax.ShapeDtypeStruct((B_ * K_, N_), jnp.float32),
                  jax.ShapeDtypeStruct((B_ * K_, C_), jnp.float32),
                  jax.ShapeDtypeStruct((B_ * N_, K_), jnp.float32)],
        scratch_types=[
            pltpu.VMEM((64,), jnp.int32),
            pltpu.VMEM((64, N_), jnp.float32),
            pltpu.VMEM((64, C_), jnp.float32),
            pltpu.VMEM((64, K_), jnp.float32),
            pltpu.SemaphoreType.DMA,
            pltpu.SemaphoreType.DMA,
            pltpu.SemaphoreType.DMA,
        ],
    )
    def k(a_hbm, hg_hbm, idx_hbm, eye_hbm, z_hbm, ar_hbm, hp_hbm, oh_hbm,
          rowabs_v, arows_v, hrows_v, zbuf, sem1, sem2, sem3):
        w = lax.axis_index("c") * _NS + lax.axis_index("s")
        b = w // 8
        pltpu.sync_copy(idx_hbm.at[pl.ds(w * 64, 64)], rowabs_v)
        for q in range(4):
            sl = pl.ds(q * 16, 16)
            rowabs_v[sl] = rowabs_v[sl] + b * N_
        cp1 = pltpu.async_copy(a_hbm.at[rowabs_v], arows_v, sem1)
        cp2 = pltpu.async_copy(hg_hbm.at[rowabs_v], hrows_v, sem2)
        pltpu.sync_copy(z_hbm, zbuf)
        pltpu.sync_copy(zbuf, oh_hbm.at[pl.ds(w * 128, 64)])
        pltpu.sync_copy(zbuf, oh_hbm.at[pl.ds(w * 128 + 64, 64)])
        cp1.wait()
        cp2.wait()
        pltpu.sync_copy(arows_v, ar_hbm.at[pl.ds(w * 64, 64)])
        pltpu.sync_copy(hrows_v, hp_hbm.at[pl.ds(w * 64, 64)])
        plsc.subcore_barrier()
        pltpu.sync_copy(eye_hbm.at[pl.ds((w % 8) * 64, 64)], zbuf)
        pltpu.async_copy(zbuf, oh_hbm.at[rowabs_v], sem3).wait()

    return k(A2, hg2, idx2, eye, zrows)


def kernel(X, A, W0a, a0a_s, a0a_n, W0b, a0b_s, a0b_n, pk,
           W1a, a1a_s, a1a_n, W1b, a1b_s, a1b_n,
           Wua, aua_s, aua_n, Wub, aub_s, aub_n,
           Wea, aea_s, aea_n, Web, aeb_s, aeb_n):
    def wf(w):
        return w.reshape(w.shape[0], HC_)

    def av(a_s, a_n):
        z = jnp.zeros((HC_, 2 * H_), jnp.float32)
        for h in range(H_):
            z = z.at[h * C_:(h + 1) * C_, h].set(a_s[h])
            z = z.at[h * C_:(h + 1) * C_, H_ + h].set(a_n[h])
        return z

    mask8, down, hg, y = _enc_call(X, A, wf(W0a), av(a0a_s, a0a_n),
                                   wf(W0b), av(a0b_s, a0b_n),
                                   pk.reshape(1, F_))

    _, idx = jax.lax.top_k(y[:, 0, :], K_)

    idxf = idx.reshape(B_ * K_)
    Ar2, hp2, oh2 = _sc_pool(A.reshape(B_ * N_, N_), hg.reshape(B_ * N_, C_),
                             idxf, jnp.eye(K_, dtype=jnp.float32),
                             jnp.zeros((64, K_), jnp.float32))

    h1b = _mid_call(hp2.reshape(B_, K_, C_), Ar2.reshape(B_, K_, N_),
                    oh2.reshape(B_, N_, K_), wf(W1a), av(a1a_s, a1a_n),
                    wf(W1b), av(a1b_s, a1b_n))

    return _dec_call(h1b, oh2.reshape(B_, N_, K_), X, down, mask8,
                     wf(Wua), av(aua_s, aua_n),
                     wf(Wub), av(aub_s, aub_n), wf(Wea[:C_]), wf(Wea[C_:]),
                     av(aea_s, aea_n), wf(Web), av(aeb_s, aeb_n))


# speedup vs baseline: 1.1021x; 1.1021x over previous
import functools

import jax
import jax.numpy as jnp
from jax import lax
from jax.experimental import pallas as pl
from jax.experimental.pallas import tpu as pltpu
from jax.experimental.pallas import tpu_sc as plsc

B_, N_, F_ = 4, 1024, 128
H_, C_ = 4, 128
HC_ = H_ * C_
K_ = 512


def _stj(astack, xp):
    return jax.lax.dot_general(astack, xp, (((0,), (1,)), ((), ())),
                               preferred_element_type=jnp.float32)


def _leaky(x):
    return jnp.maximum(x, 0.2 * x)


def _heads(maskf, st, xp, n):
    outs = []
    for h in range(H_):
        s = st[h]
        t = st[H_ + h]
        mhat = _leaky(s + jnp.max(t))
        a = (s - mhat)[:, None] + t[None, :]
        b2 = (0.2 * s - mhat)[:, None] + (0.2 * t)[None, :]
        p = jnp.exp(jnp.maximum(a, b2)) * maskf
        v = jnp.concatenate(
            [xp[:, h * C_:(h + 1) * C_], jnp.ones((n, 128), jnp.float32)],
            axis=1)
        oh = jnp.dot(p, v, preferred_element_type=jnp.float32)
        outs.append(oh[:, :C_] * (1.0 / oh[:, C_:C_ + 1]))
    return outs


def _gat_cat(maskf, st, xp, n):
    return jnp.concatenate(_heads(maskf, st, xp, n), axis=1)


def _gat_mean(maskf, st, xp, n):
    o = _heads(maskf, st, xp, n)
    return jnp.maximum((o[0] + o[1] + o[2] + o[3]) * (1.0 / H_), 0.0)


def _diag_mask(pos, n):
    r = jax.lax.broadcasted_iota(jnp.int32, (n, n), 0)
    c = jax.lax.broadcasted_iota(jnp.int32, (n, n), 1)
    return pos | (r == c)


def _enc_body(x_ref, adj_ref, w0a_ref, a0a_ref, w0b_ref, a0b_ref, pk_ref,
              m_ref, d_ref, g_ref, y_ref):
    m = _diag_mask(adj_ref[0] > 0, N_)
    m_ref[0] = m.astype(jnp.int8)
    maskf = m.astype(jnp.float32)
    xp = jnp.dot(x_ref[0], w0a_ref[...], preferred_element_type=jnp.float32)
    h0a = _gat_cat(maskf, _stj(a0a_ref[...], xp), xp, N_)
    xp = jnp.dot(h0a, w0b_ref[...], preferred_element_type=jnp.float32)
    h = _gat_mean(maskf, _stj(a0b_ref[...], xp), xp, N_)
    d_ref[0] = h
    pk = pk_ref[0]
    kn = pk / (jnp.sqrt(jnp.sum(pk * pk)) + 1e-12)
    y = jnp.sum(h * kn[None, :], axis=1)
    y_ref[0, 0, :] = y
    g_ref[0] = h * jnp.tanh(y)[:, None]


def _enc_call(X, A, w0a, a0a, w0b, a0b, pk):
    return pl.pallas_call(
        _enc_body, grid=(B_,),
        in_specs=[
            pl.BlockSpec((1, N_, F_), lambda i: (i, 0, 0)),
            pl.BlockSpec((1, N_, N_), lambda i: (i, 0, 0)),
            pl.BlockSpec((F_, HC_), lambda i: (0, 0)),
            pl.BlockSpec((HC_, 8), lambda i: (0, 0)),
            pl.BlockSpec((HC_, HC_), lambda i: (0, 0)),
            pl.BlockSpec((HC_, 8), lambda i: (0, 0)),
            pl.BlockSpec((1, C_), lambda i: (0, 0)),
        ],
        out_specs=[
            pl.BlockSpec((1, N_, N_), lambda i: (i, 0, 0)),
            pl.BlockSpec((1, N_, C_), lambda i: (i, 0, 0)),
            pl.BlockSpec((1, N_, C_), lambda i: (i, 0, 0)),
            pl.BlockSpec((1, 1, N_), lambda i: (i, 0, 0)),
        ],
        out_shape=[
            jax.ShapeDtypeStruct((B_, N_, N_), jnp.int8),
            jax.ShapeDtypeStruct((B_, N_, C_), jnp.float32),
            jax.ShapeDtypeStruct((B_, N_, C_), jnp.float32),
            jax.ShapeDtypeStruct((B_, 1, N_), jnp.float32),
        ],
    )(X, A, w0a, a0a, w0b, a0b, pk)


def _mid_body(hp_ref, ar_ref, oh_ref, w1a_ref, a1a_ref, w1b_ref, a1b_ref,
              o_ref):
    ap = jnp.dot(ar_ref[0], oh_ref[0], preferred_element_type=jnp.float32)
    maskf = _diag_mask(ap > 0, K_).astype(jnp.float32)
    xp = jnp.dot(hp_ref[0], w1a_ref[...], preferred_element_type=jnp.float32)
    h1a = _gat_cat(maskf, _stj(a1a_ref[...], xp), xp, K_)
    xp = jnp.dot(h1a, w1b_ref[...], preferred_element_type=jnp.float32)
    o_ref[0] = _gat_mean(maskf, _stj(a1b_ref[...], xp), xp, K_)


def _mid_call(hp, Ar, oh, w1a, a1a, w1b, a1b):
    return pl.pallas_call(
        _mid_body, grid=(B_,),
        in_specs=[
            pl.BlockSpec((1, K_, C_), lambda i: (i, 0, 0)),
            pl.BlockSpec((1, K_, N_), lambda i: (i, 0, 0)),
            pl.BlockSpec((1, N_, K_), lambda i: (i, 0, 0)),
            pl.BlockSpec((C_, HC_), lambda i: (0, 0)),
            pl.BlockSpec((HC_, 8), lambda i: (0, 0)),
            pl.BlockSpec((HC_, HC_), lambda i: (0, 0)),
            pl.BlockSpec((HC_, 8), lambda i: (0, 0)),
        ],
        out_specs=pl.BlockSpec((1, K_, C_), lambda i: (i, 0, 0)),
        out_shape=jax.ShapeDtypeStruct((B_, K_, C_), jnp.float32),
    )(hp, Ar, oh, w1a, a1a, w1b, a1b)


def _dec_body(hu0_ref, x_ref, d_ref, m_ref, wua_ref, aua_ref, wub_ref,
              aub_ref, weah_ref, weal_ref, aea_ref, web_ref, aeb_ref, o_ref):
    maskf = m_ref[0].astype(jnp.float32)
    xp = jnp.dot(hu0_ref[0], wua_ref[...], preferred_element_type=jnp.float32)
    hua = _gat_cat(maskf, _stj(aua_ref[...], xp), xp, N_)
    xp = jnp.dot(hua, wub_ref[...], preferred_element_type=jnp.float32)
    hu = _gat_mean(maskf, _stj(aub_ref[...], xp), xp, N_) + d_ref[0]
    xp = jnp.dot(hu, weah_ref[...], preferred_element_type=jnp.float32)
    xp = xp + jnp.dot(x_ref[0], weal_ref[...],
                      preferred_element_type=jnp.float32)
    hea = _gat_cat(maskf, _stj(aea_ref[...], xp), xp, N_)
    xp = jnp.dot(hea, web_ref[...], preferred_element_type=jnp.float32)
    o_ref[0] = _gat_mean(maskf, _stj(aeb_ref[...], xp), xp, N_)


def _dec_call(hu0, X, down, mask8, wua, aua, wub, aub, weah, weal, aea,
              web, aeb):
    return pl.pallas_call(
        _dec_body, grid=(B_,),
        in_specs=[
            pl.BlockSpec((1, N_, C_), lambda i: (i, 0, 0)),
            pl.BlockSpec((1, N_, F_), lambda i: (i, 0, 0)),
            pl.BlockSpec((1, N_, C_), lambda i: (i, 0, 0)),
            pl.BlockSpec((1, N_, N_), lambda i: (i, 0, 0)),
            pl.BlockSpec((C_, HC_), lambda i: (0, 0)),
            pl.BlockSpec((HC_, 8), lambda i: (0, 0)),
            pl.BlockSpec((HC_, HC_), lambda i: (0, 0)),
            pl.BlockSpec((HC_, 8), lambda i: (0, 0)),
            pl.BlockSpec((C_, HC_), lambda i: (0, 0)),
            pl.BlockSpec((F_, HC_), lambda i: (0, 0)),
            pl.BlockSpec((HC_, 8), lambda i: (0, 0)),
            pl.BlockSpec((HC_, HC_), lambda i: (0, 0)),
            pl.BlockSpec((HC_, 8), lambda i: (0, 0)),
        ],
        out_specs=pl.BlockSpec((1, N_, C_), lambda i: (i, 0, 0)),
        out_shape=jax.ShapeDtypeStruct((B_, N_, C_), jnp.float32),
    )(hu0, X, down, mask8, wua, aua, wub, aub, weah, weal, aea, web, aeb)


_NC, _NS = 2, 16


def _sc_pool(A2, hg2, idx2, eye, zrows):
    mesh = plsc.VectorSubcoreMesh(core_axis_name="c", subcore_axis_name="s")

    @functools.partial(
        pl.kernel, mesh=mesh,
        out_type=[jax.ShapeDtypeStruct((B_ * K_, N_), jnp.float32),
                  jax.ShapeDtypeStruct((B_ * K_, C_), jnp.float32),
                  jax.ShapeDtypeStruct((B_ * N_, K_), jnp.float32)],
        scratch_types=[
            pltpu.VMEM((64,), jnp.int32),
            pltpu.VMEM((64, N_), jnp.float32),
            pltpu.VMEM((64, C_), jnp.float32),
            pltpu.VMEM((64, K_), jnp.float32),
            pltpu.SemaphoreType.DMA,
            pltpu.SemaphoreType.DMA,
            pltpu.SemaphoreType.DMA,
        ],
    )
    def k(a_hbm, hg_hbm, idx_hbm, eye_hbm, z_hbm, ar_hbm, hp_hbm, oh_hbm,
          rowabs_v, arows_v, hrows_v, zbuf, sem1, sem2, sem3):
        w = lax.axis_index("c") * _NS + lax.axis_index("s")
        b = w // 8
        pltpu.sync_copy(idx_hbm.at[pl.ds(w * 64, 64)], rowabs_v)
        for q in range(4):
            sl = pl.ds(q * 16, 16)
            rowabs_v[sl] = rowabs_v[sl] + b * N_
        cp1 = pltpu.async_copy(a_hbm.at[rowabs_v], arows_v, sem1)
        cp2 = pltpu.async_copy(hg_hbm.at[rowabs_v], hrows_v, sem2)
        pltpu.sync_copy(z_hbm, zbuf)
        pltpu.sync_copy(zbuf, oh_hbm.at[pl.ds(w * 128, 64)])
        pltpu.sync_copy(zbuf, oh_hbm.at[pl.ds(w * 128 + 64, 64)])
        cp1.wait()
        cp2.wait()
        pltpu.sync_copy(arows_v, ar_hbm.at[pl.ds(w * 64, 64)])
        pltpu.sync_copy(hrows_v, hp_hbm.at[pl.ds(w * 64, 64)])
        plsc.subcore_barrier()
        pltpu.sync_copy(eye_hbm.at[pl.ds((w % 8) * 64, 64)], zbuf)
        pltpu.async_copy(zbuf, oh_hbm.at[rowabs_v], sem3).wait()

    return k(A2, hg2, idx2, eye, zrows)


def _sc_unpool(h1b2, idx2, zrows):
    mesh = plsc.VectorSubcoreMesh(core_axis_name="c", subcore_axis_name="s")

    @functools.partial(
        pl.kernel, mesh=mesh,
        out_type=jax.ShapeDtypeStruct((B_ * N_, C_), jnp.float32),
        scratch_types=[
            pltpu.VMEM((64, C_), jnp.float32),
            pltpu.VMEM((32, C_), jnp.float32),
            pltpu.VMEM((32,), jnp.int32),
            pltpu.SemaphoreType.DMA,
        ],
    )
    def k(h_hbm, idx_hbm, z_hbm, out_hbm, zbuf, rbuf, iabs, sem):
        c = lax.axis_index("c")
        sid = lax.axis_index("s")
        pltpu.sync_copy(z_hbm, zbuf)
        for bb in range(2):
            b = c * 2 + bb
            pltpu.sync_copy(zbuf, out_hbm.at[pl.ds(b * N_ + sid * 64, 64)])
        plsc.subcore_barrier()
        for bb in range(2):
            b = c * 2 + bb
            base = b * K_ + sid * 32
            pltpu.sync_copy(idx_hbm.at[pl.ds(base, 32)], iabs)
            for q in range(2):
                sl = pl.ds(q * 16, 16)
                iabs[sl] = iabs[sl] + b * N_
            pltpu.sync_copy(h_hbm.at[pl.ds(base, 32)], rbuf)
            pltpu.async_copy(rbuf, out_hbm.at[iabs], sem).wait()

    return k(h1b2, idx2, zrows)


def kernel(X, A, W0a, a0a_s, a0a_n, W0b, a0b_s, a0b_n, pk,
           W1a, a1a_s, a1a_n, W1b, a1b_s, a1b_n,
           Wua, aua_s, aua_n, Wub, aub_s, aub_n,
           Wea, aea_s, aea_n, Web, aeb_s, aeb_n):
    def wf(w):
        return w.reshape(w.shape[0], HC_)

    def av(a_s, a_n):
        z = jnp.zeros((HC_, 2 * H_), jnp.float32)
        for h in range(H_):
            z = z.at[h * C_:(h + 1) * C_, h].set(a_s[h])
            z = z.at[h * C_:(h + 1) * C_, H_ + h].set(a_n[h])
        return z

    mask8, down, hg, y = _enc_call(X, A, wf(W0a), av(a0a_s, a0a_n),
                                   wf(W0b), av(a0b_s, a0b_n),
                                   pk.reshape(1, F_))

    _, idx = jax.lax.top_k(y[:, 0, :], K_)

    idxf = idx.reshape(B_ * K_)
    Ar2, hp2, oh2 = _sc_pool(A.reshape(B_ * N_, N_), hg.reshape(B_ * N_, C_),
                             idxf, jnp.eye(K_, dtype=jnp.float32),
                             jnp.zeros((64, K_), jnp.float32))

    h1b = _mid_call(hp2.reshape(B_, K_, C_), Ar2.reshape(B_, K_, N_),
                    oh2.reshape(B_, N_, K_), wf(W1a), av(a1a_s, a1a_n),
                    wf(W1b), av(a1b_s, a1b_n))

    hu0 = _sc_unpool(h1b.reshape(B_ * K_, C_), idxf,
                     jnp.zeros((64, C_), jnp.float32)).reshape(B_, N_, C_)

    return _dec_call(hu0, X, down, mask8, wf(Wua), av(aua_s, aua_n),
                     wf(Wub), av(aub_s, aub_n), wf(Wea[:C_]), wf(Wea[C_:]),
                     av(aea_s, aea_n), wf(Web), av(aeb_s, aeb_n))
